# baseline (speedup 1.0000x reference)
import functools

import jax
import jax.numpy as jnp
from jax import lax
from jax.experimental import pallas as pl
from jax.experimental.pallas import tpu as pltpu
from jax.experimental.pallas import tpu_sc as plsc

N = 10000
E = 320000
IN_CH, HID1, HID2, OUT_CH = 128, 64, 32, 32

NC, NS = 2, 16
NW = NC * NS
EPW = 10752
E_PAD = NW * EPW
N_PAD = 10112
RPT = N_PAD // NS

_mesh = plsc.VectorSubcoreMesh(core_axis_name="c", subcore_axis_name="s")


def _make_sc_edge_pass(D, NBUF, C, CH):
    assert C * CH == EPW and CH % NBUF == 0

    @functools.partial(
        pl.kernel,
        out_type=jax.ShapeDtypeStruct((NC, N_PAD, D), jnp.float32),
        mesh=_mesh,
        scratch_types=[
            pltpu.VMEM((CH, C), jnp.int32),
            pltpu.VMEM((CH, C), jnp.int32),
            [pltpu.VMEM((C, D), jnp.float32)] * NBUF,
            pltpu.VMEM_SHARED((N, D), jnp.float32),
            pltpu.VMEM_SHARED((N_PAD, D), jnp.float32),
            [pltpu.SemaphoreType.DMA] * NBUF,
            [pltpu.SemaphoreType.DMA] * NBUF,
        ],
        compiler_params=pltpu.CompilerParams(use_tc_tiling_on_sc=False, skip_device_barrier=True),
    )
    def k(hs_hbm, src_hbm, dst_hbm, zeros_hbm, out_hbm,
          src_v, dst_v, rows, hs_sh, acc_sh, gsem, ssem):
        cid = lax.axis_index("c")
        sid = lax.axis_index("s")
        wid = sid * NC + cid
        pltpu.sync_copy(src_hbm.at[wid], src_v)
        pltpu.sync_copy(dst_hbm.at[wid], dst_v)
        SR = 624
        pltpu.sync_copy(hs_hbm.at[pl.ds(sid * SR, SR)],
                        hs_sh.at[pl.ds(sid * SR, SR)])

        @pl.when(sid == NS - 1)
        def _():
            pltpu.sync_copy(hs_hbm.at[pl.ds(NS * SR, N - NS * SR)],
                            hs_sh.at[pl.ds(NS * SR, N - NS * SR)])

        pltpu.sync_copy(zeros_hbm, acc_sh.at[pl.ds(sid * RPT, RPT)])
        plsc.subcore_barrier()

        for b in range(NBUF):
            pltpu.async_copy(hs_sh.at[src_v.at[b]], rows[b], gsem[b])

        def body(p, carry):
            j0 = p * NBUF
            for b in range(NBUF):
                j = j0 + b
                pltpu.make_async_copy(
                    hs_sh.at[src_v.at[j]], rows[b], gsem[b]).wait()
                pltpu.async_copy(
                    rows[b], acc_sh.at[dst_v.at[j]], ssem[b], add=True)

                @pl.when(j + NBUF < CH)
                def _():
                    pltpu.make_async_copy(
                        rows[b], acc_sh.at[dst_v.at[j]], ssem[b]).wait()
                    pltpu.async_copy(
                        hs_sh.at[src_v.at[j + NBUF]], rows[b], gsem[b])

            return carry

        lax.fori_loop(0, CH // NBUF, body, 0)
        for b in range(NBUF):
            pltpu.make_async_copy(
                rows[b], acc_sh.at[dst_v.at[CH - NBUF + b]], ssem[b]).wait()
        plsc.subcore_barrier()
        pltpu.sync_copy(acc_sh.at[pl.ds(sid * RPT, RPT)],
                        out_hbm.at[cid, pl.ds(sid * RPT, RPT)])

    return k


DEG_W = 8
DC = 256
DCH = EPW // DC


@functools.partial(
    pl.kernel,
    out_type=jax.ShapeDtypeStruct((NC, N_PAD, DEG_W), jnp.float32),
    mesh=_mesh,
    scratch_types=[
        pltpu.VMEM((DCH, DC), jnp.int32),
        pltpu.VMEM((DC, DEG_W), jnp.float32),
        pltpu.VMEM_SHARED((N_PAD, DEG_W), jnp.float32),
        pltpu.SemaphoreType.DMA,
    ],
    compiler_params=pltpu.CompilerParams(use_tc_tiling_on_sc=False, skip_device_barrier=True),
)
def _sc_degree(dst_hbm, ones_hbm, zeros_hbm, out_hbm, dst_v, ones_v, acc_sh,
               sem):
    cid = lax.axis_index("c")
    sid = lax.axis_index("s")
    wid = sid * NC + cid
    pltpu.sync_copy(dst_hbm.at[wid], dst_v)
    pltpu.sync_copy(ones_hbm, ones_v)
    pltpu.sync_copy(zeros_hbm, acc_sh.at[pl.ds(sid * RPT, RPT)])
    plsc.subcore_barrier()

    DK = 14

    def body(p, carry):
        j0 = p * DK
        for b in range(DK):
            pltpu.async_copy(ones_v, acc_sh.at[dst_v.at[j0 + b]], sem,
                             add=True)
        for b in range(DK):
            pltpu.make_async_copy(ones_v, acc_sh.at[dst_v.at[j0 + b]],
                                  sem).wait()
        return carry

    lax.fori_loop(0, DCH // DK, body, 0)
    plsc.subcore_barrier()
    pltpu.sync_copy(acc_sh.at[pl.ds(sid * RPT, RPT)],
                    out_hbm.at[cid, pl.ds(sid * RPT, RPT)])


ROWS = 2000
_DOT = functools.partial(jnp.dot, precision=lax.Precision.HIGHEST,
                         preferred_element_type=jnp.float32)


def _tc_xw1(x, W1):
    def body(x_ref, w_ref, out_ref):
        out_ref[...] = _DOT(x_ref[...], w_ref[...])

    return pl.pallas_call(
        body,
        grid=(N // ROWS,),
        in_specs=[
            pl.BlockSpec((ROWS, IN_CH), lambda i: (i, 0)),
            pl.BlockSpec((IN_CH, HID1), lambda i: (0, 0)),
        ],
        out_specs=pl.BlockSpec((ROWS, HID1), lambda i: (i, 0)),
        out_shape=jax.ShapeDtypeStruct((N, HID1), jnp.float32),
    )(x, W1)


def _tc_first(deg, xw1):
    def body(deg_ref, xw_ref, dinv_ref, hs_ref):
        d = deg_ref[0, :, 0:1] + deg_ref[1, :, 0:1] + 1.0
        dinv = lax.rsqrt(d)
        dinv_ref[...] = dinv
        hs_ref[...] = dinv * xw_ref[...]

    return pl.pallas_call(
        body,
        grid=(N // ROWS,),
        in_specs=[
            pl.BlockSpec((NC, ROWS, DEG_W), lambda i: (0, i, 0)),
            pl.BlockSpec((ROWS, HID1), lambda i: (i, 0)),
        ],
        out_specs=[
            pl.BlockSpec((ROWS, 1), lambda i: (i, 0)),
            pl.BlockSpec((ROWS, HID1), lambda i: (i, 0)),
        ],
        out_shape=[
            jax.ShapeDtypeStruct((N, 1), jnp.float32),
            jax.ShapeDtypeStruct((N, HID1), jnp.float32),
        ],
    )(deg, xw1)


def _tc_mid(acc, hs, dinv, b, W_next, D, D_next):
    def body(acc_ref, hs_ref, dinv_ref, b_ref, w_ref, out_ref):
        a = acc_ref[0] + acc_ref[1] + hs_ref[...]
        h = jnp.maximum(dinv_ref[...] * a + b_ref[...], 0.0)
        out_ref[...] = dinv_ref[...] * _DOT(h, w_ref[...])

    return pl.pallas_call(
        body,
        grid=(N // ROWS,),
        in_specs=[
            pl.BlockSpec((NC, ROWS, D), lambda i: (0, i, 0)),
            pl.BlockSpec((ROWS, D), lambda i: (i, 0)),
            pl.BlockSpec((ROWS, 1), lambda i: (i, 0)),
            pl.BlockSpec((1, D), lambda i: (0, 0)),
            pl.BlockSpec((D, D_next), lambda i: (0, 0)),
        ],
        out_specs=pl.BlockSpec((ROWS, D_next), lambda i: (i, 0)),
        out_shape=jax.ShapeDtypeStruct((N, D_next), jnp.float32),
    )(acc, hs, dinv, b, W_next)


def _tc_last(acc, hs, dinv, b):
    def body(acc_ref, hs_ref, dinv_ref, b_ref, out_ref):
        a = acc_ref[0] + acc_ref[1] + hs_ref[...]
        out_ref[...] = dinv_ref[...] * a + b_ref[...]

    return pl.pallas_call(
        body,
        grid=(N // ROWS,),
        in_specs=[
            pl.BlockSpec((NC, ROWS, OUT_CH), lambda i: (0, i, 0)),
            pl.BlockSpec((ROWS, OUT_CH), lambda i: (i, 0)),
            pl.BlockSpec((ROWS, 1), lambda i: (i, 0)),
            pl.BlockSpec((1, OUT_CH), lambda i: (0, 0)),
        ],
        out_specs=pl.BlockSpec((ROWS, OUT_CH), lambda i: (i, 0)),
        out_shape=jax.ShapeDtypeStruct((N, OUT_CH), jnp.float32),
    )(acc, hs, dinv, b)


_sc_edge64 = _make_sc_edge_pass(HID1, 3, 128, 84)
_sc_edge32 = _make_sc_edge_pass(HID2, 7, 256, 42)


def kernel(x, edge_index, W1, b1, W2, b2, W3, b3):
    pad = E_PAD - E
    src_p = jnp.concatenate(
        [edge_index[0], jnp.zeros((pad,), jnp.int32)]).reshape(NW, 84, 128)
    dst_p = jnp.concatenate(
        [edge_index[1], jnp.full((pad,), N, jnp.int32)]).reshape(NW, 84, 128)
    src_w = src_p.reshape(NW, 42, 256)
    dst_w = dst_p.reshape(NW, 42, 256)

    ones8 = jnp.ones((DC, DEG_W), jnp.float32)
    zeros8 = jnp.zeros((RPT, DEG_W), jnp.float32)
    zeros64 = jnp.zeros((RPT, HID1), jnp.float32)
    zeros32 = jnp.zeros((RPT, HID2), jnp.float32)

    deg = _sc_degree(dst_w, ones8, zeros8)
    xw1 = _tc_xw1(x, W1)

    dinv, hs1 = _tc_first(deg, xw1)
    acc1 = _sc_edge64(hs1, src_p, dst_p, zeros64)
    hs2 = _tc_mid(acc1, hs1, dinv, b1[None, :], W2, HID1, HID2)
    acc2 = _sc_edge32(hs2, src_w, dst_w, zeros32)
    hs3 = _tc_mid(acc2, hs2, dinv, b2[None, :], W3, HID2, OUT_CH)
    acc3 = _sc_edge32(hs3, src_w, dst_w, zeros32)
    return _tc_last(acc3, hs3, dinv, b3[None, :])


# speedup vs baseline: 1.0004x; 1.0004x over previous
import functools

import jax
import jax.numpy as jnp
from jax import lax
from jax.experimental import pallas as pl
from jax.experimental.pallas import tpu as pltpu
from jax.experimental.pallas import tpu_sc as plsc

N = 10000
E = 320000
IN_CH, HID1, HID2, OUT_CH = 128, 64, 32, 32

NC, NS = 2, 16
NW = NC * NS
EPW = 10752
E_PAD = NW * EPW
N_PAD = 10112
RPT = N_PAD // NS

_mesh = plsc.VectorSubcoreMesh(core_axis_name="c", subcore_axis_name="s")


def _make_sc_edge_pass(D, NBUF, C, CH):
    assert C * CH == EPW and CH % NBUF == 0

    @functools.partial(
        pl.kernel,
        out_type=jax.ShapeDtypeStruct((NC, N_PAD, D), jnp.float32),
        mesh=_mesh,
        scratch_types=[
            pltpu.VMEM((CH, C), jnp.int32),
            pltpu.VMEM((CH, C), jnp.int32),
            [pltpu.VMEM((C, D), jnp.float32)] * NBUF,
            pltpu.VMEM_SHARED((N, D), jnp.float32),
            pltpu.VMEM_SHARED((N_PAD, D), jnp.float32),
            [pltpu.SemaphoreType.DMA] * NBUF,
            [pltpu.SemaphoreType.DMA] * NBUF,
        ],
        compiler_params=pltpu.CompilerParams(use_tc_tiling_on_sc=False),
    )
    def k(hs_hbm, src_hbm, dst_hbm, zeros_hbm, out_hbm,
          src_v, dst_v, rows, hs_sh, acc_sh, gsem, ssem):
        cid = lax.axis_index("c")
        sid = lax.axis_index("s")
        wid = sid * NC + cid
        pltpu.sync_copy(src_hbm.at[wid], src_v)
        pltpu.sync_copy(dst_hbm.at[wid], dst_v)
        SR = 624
        pltpu.sync_copy(hs_hbm.at[pl.ds(sid * SR, SR)],
                        hs_sh.at[pl.ds(sid * SR, SR)])

        @pl.when(sid == NS - 1)
        def _():
            pltpu.sync_copy(hs_hbm.at[pl.ds(NS * SR, N - NS * SR)],
                            hs_sh.at[pl.ds(NS * SR, N - NS * SR)])

        pltpu.sync_copy(zeros_hbm, acc_sh.at[pl.ds(sid * RPT, RPT)])
        plsc.subcore_barrier()

        for b in range(NBUF):
            pltpu.async_copy(hs_sh.at[src_v.at[b]], rows[b], gsem[b])

        def body(p, carry):
            j0 = p * NBUF
            for b in range(NBUF):
                j = j0 + b
                pltpu.make_async_copy(
                    hs_sh.at[src_v.at[j]], rows[b], gsem[b]).wait()
                pltpu.async_copy(
                    rows[b], acc_sh.at[dst_v.at[j]], ssem[b], add=True)

                @pl.when(j + NBUF < CH)
                def _():
                    pltpu.make_async_copy(
                        rows[b], acc_sh.at[dst_v.at[j]], ssem[b]).wait()
                    pltpu.async_copy(
                        hs_sh.at[src_v.at[j + NBUF]], rows[b], gsem[b])

            return carry

        lax.fori_loop(0, CH // NBUF, body, 0)
        for b in range(NBUF):
            pltpu.make_async_copy(
                rows[b], acc_sh.at[dst_v.at[CH - NBUF + b]], ssem[b]).wait()
        plsc.subcore_barrier()
        pltpu.sync_copy(acc_sh.at[pl.ds(sid * RPT, RPT)],
                        out_hbm.at[cid, pl.ds(sid * RPT, RPT)])

    return k


DEG_W = 8
DC = 256
DCH = EPW // DC


@functools.partial(
    pl.kernel,
    out_type=jax.ShapeDtypeStruct((NC, N_PAD, DEG_W), jnp.float32),
    mesh=_mesh,
    scratch_types=[
        pltpu.VMEM((DCH, DC), jnp.int32),
        pltpu.VMEM((DC, DEG_W), jnp.float32),
        pltpu.VMEM_SHARED((N_PAD, DEG_W), jnp.float32),
        pltpu.SemaphoreType.DMA,
    ],
    compiler_params=pltpu.CompilerParams(use_tc_tiling_on_sc=False),
)
def _sc_degree(dst_hbm, ones_hbm, zeros_hbm, out_hbm, dst_v, ones_v, acc_sh,
               sem):
    cid = lax.axis_index("c")
    sid = lax.axis_index("s")
    wid = sid * NC + cid
    pltpu.sync_copy(dst_hbm.at[wid], dst_v)
    pltpu.sync_copy(ones_hbm, ones_v)
    pltpu.sync_copy(zeros_hbm, acc_sh.at[pl.ds(sid * RPT, RPT)])
    plsc.subcore_barrier()

    DK = 14

    def body(p, carry):
        j0 = p * DK
        for b in range(DK):
            pltpu.async_copy(ones_v, acc_sh.at[dst_v.at[j0 + b]], sem,
                             add=True)
        for b in range(DK):
            pltpu.make_async_copy(ones_v, acc_sh.at[dst_v.at[j0 + b]],
                                  sem).wait()
        return carry

    lax.fori_loop(0, DCH // DK, body, 0)
    plsc.subcore_barrier()
    pltpu.sync_copy(acc_sh.at[pl.ds(sid * RPT, RPT)],
                    out_hbm.at[cid, pl.ds(sid * RPT, RPT)])


ROWS = 2000
_DOT = functools.partial(jnp.dot, precision=lax.Precision.HIGHEST,
                         preferred_element_type=jnp.float32)


def _tc_xw1(x, W1):
    def body(x_ref, w_ref, out_ref):
        out_ref[...] = _DOT(x_ref[...], w_ref[...])

    return pl.pallas_call(
        body,
        grid=(N // ROWS,),
        in_specs=[
            pl.BlockSpec((ROWS, IN_CH), lambda i: (i, 0)),
            pl.BlockSpec((IN_CH, HID1), lambda i: (0, 0)),
        ],
        out_specs=pl.BlockSpec((ROWS, HID1), lambda i: (i, 0)),
        out_shape=jax.ShapeDtypeStruct((N, HID1), jnp.float32),
    )(x, W1)


def _tc_first(deg, xw1):
    def body(deg_ref, xw_ref, dinv_ref, hs_ref):
        d = deg_ref[0, :, 0:1] + deg_ref[1, :, 0:1] + 1.0
        dinv = lax.rsqrt(d)
        dinv_ref[...] = dinv
        hs_ref[...] = dinv * xw_ref[...]

    return pl.pallas_call(
        body,
        grid=(N // ROWS,),
        in_specs=[
            pl.BlockSpec((NC, ROWS, DEG_W), lambda i: (0, i, 0)),
            pl.BlockSpec((ROWS, HID1), lambda i: (i, 0)),
        ],
        out_specs=[
            pl.BlockSpec((ROWS, 1), lambda i: (i, 0)),
            pl.BlockSpec((ROWS, HID1), lambda i: (i, 0)),
        ],
        out_shape=[
            jax.ShapeDtypeStruct((N, 1), jnp.float32),
            jax.ShapeDtypeStruct((N, HID1), jnp.float32),
        ],
    )(deg, xw1)


def _tc_mid(acc, hs, dinv, b, W_next, D, D_next):
    def body(acc_ref, hs_ref, dinv_ref, b_ref, w_ref, out_ref):
        a = acc_ref[0] + acc_ref[1] + hs_ref[...]
        h = jnp.maximum(dinv_ref[...] * a + b_ref[...], 0.0)
        out_ref[...] = dinv_ref[...] * _DOT(h, w_ref[...])

    return pl.pallas_call(
        body,
        grid=(N // ROWS,),
        in_specs=[
            pl.BlockSpec((NC, ROWS, D), lambda i: (0, i, 0)),
            pl.BlockSpec((ROWS, D), lambda i: (i, 0)),
            pl.BlockSpec((ROWS, 1), lambda i: (i, 0)),
            pl.BlockSpec((1, D), lambda i: (0, 0)),
            pl.BlockSpec((D, D_next), lambda i: (0, 0)),
        ],
        out_specs=pl.BlockSpec((ROWS, D_next), lambda i: (i, 0)),
        out_shape=jax.ShapeDtypeStruct((N, D_next), jnp.float32),
    )(acc, hs, dinv, b, W_next)


def _tc_last(acc, hs, dinv, b):
    def body(acc_ref, hs_ref, dinv_ref, b_ref, out_ref):
        a = acc_ref[0] + acc_ref[1] + hs_ref[...]
        out_ref[...] = dinv_ref[...] * a + b_ref[...]

    return pl.pallas_call(
        body,
        grid=(N // ROWS,),
        in_specs=[
            pl.BlockSpec((NC, ROWS, OUT_CH), lambda i: (0, i, 0)),
            pl.BlockSpec((ROWS, OUT_CH), lambda i: (i, 0)),
            pl.BlockSpec((ROWS, 1), lambda i: (i, 0)),
            pl.BlockSpec((1, OUT_CH), lambda i: (0, 0)),
        ],
        out_specs=pl.BlockSpec((ROWS, OUT_CH), lambda i: (i, 0)),
        out_shape=jax.ShapeDtypeStruct((N, OUT_CH), jnp.float32),
    )(acc, hs, dinv, b)


_sc_edge64 = _make_sc_edge_pass(HID1, 3, 128, 84)
_sc_edge32 = _make_sc_edge_pass(HID2, 7, 256, 42)


def kernel(x, edge_index, W1, b1, W2, b2, W3, b3):
    pad = E_PAD - E
    src_p = jnp.concatenate(
        [edge_index[0], jnp.zeros((pad,), jnp.int32)]).reshape(NW, 84, 128)
    dst_p = jnp.concatenate(
        [edge_index[1], jnp.full((pad,), N, jnp.int32)]).reshape(NW, 84, 128)
    src_w = src_p.reshape(NW, 42, 256)
    dst_w = dst_p.reshape(NW, 42, 256)

    ones8 = jnp.ones((DC, DEG_W), jnp.float32)
    zeros8 = jnp.zeros((RPT, DEG_W), jnp.float32)
    zeros64 = jnp.zeros((RPT, HID1), jnp.float32)
    zeros32 = jnp.zeros((RPT, HID2), jnp.float32)

    deg = _sc_degree(dst_w, ones8, zeros8)
    xw1 = _tc_xw1(x, W1)

    dinv, hs1 = _tc_first(deg, xw1)
    acc1 = _sc_edge64(hs1, src_p, dst_p, zeros64)
    hs2 = _tc_mid(acc1, hs1, dinv, b1[None, :], W2, HID1, HID2)
    acc2 = _sc_edge32(hs2, src_w, dst_w, zeros32)
    hs3 = _tc_mid(acc2, hs2, dinv, b2[None, :], W3, HID2, OUT_CH)
    acc3 = _sc_edge32(hs3, src_w, dst_w, zeros32)
    return _tc_last(acc3, hs3, dinv, b3[None, :])
